# Initial kernel scaffold; baseline (speedup 1.0000x reference)
import jax
import jax.numpy as jnp
from jax.experimental import pallas as pl


def kernel(x, edge_index, W1, b1, W2, b2, W3, b3):
    raise NotImplementedError("write your pallas kernel here")


# speedup vs baseline: 4.8308x; 4.8308x over previous
import functools

import jax
import jax.numpy as jnp
from jax import lax
from jax.experimental import pallas as pl
from jax.experimental.pallas import tpu as pltpu
from jax.experimental.pallas import tpu_sc as plsc

N_NODES = 10000
D = 128
N_EDGES = 320000

NUM_CORES = 2
NUM_SUBCORES = 16

NPAD = 10240
EROWS = 2560
EPAD = EROWS * 128
HALF = NPAD // 2
ACC_ROWS = HALF + 8
TRASH = HALF

DEG_ROWS_PER_TILE = EROWS // (NUM_CORES * NUM_SUBCORES)
AGG_ROWS_PER_TILE = EROWS // NUM_SUBCORES
DEG_WB = NPAD // NUM_SUBCORES
AGG_WB = HALF // NUM_SUBCORES

BLK = 1280
GRID = NPAD // BLK

_mesh = plsc.VectorSubcoreMesh(core_axis_name="c", subcore_axis_name="s")


@functools.partial(
    pl.kernel,
    out_type=jax.ShapeDtypeStruct((NUM_CORES, NPAD, D), jnp.float32),
    mesh=_mesh,
    scratch_types=[
        pltpu.VMEM((DEG_ROWS_PER_TILE, 128), jnp.int32),
        pltpu.VMEM((128, D), jnp.float32),
        pltpu.VMEM_SHARED((NPAD, D), jnp.float32),
        pltpu.SemaphoreType.DMA,
    ],
)
def _sc_degree(dst_hbm, z_hbm, ones_hbm, out_hbm, dsti, ones_v, acc_sh, sem):
    cid = lax.axis_index("c")
    sid = lax.axis_index("s")
    base = (cid * NUM_SUBCORES + sid) * DEG_ROWS_PER_TILE
    pltpu.sync_copy(dst_hbm.at[pl.ds(base, DEG_ROWS_PER_TILE)], dsti)
    pltpu.sync_copy(ones_hbm, ones_v)
    wb = sid * DEG_WB
    pltpu.sync_copy(z_hbm.at[pl.ds(0, DEG_WB)], acc_sh.at[pl.ds(wb, DEG_WB)])
    plsc.subcore_barrier()

    @pl.loop(0, DEG_ROWS_PER_TILE, step=8)
    def _(c0):
        for b in range(8):
            pltpu.sync_copy(ones_v, acc_sh.at[dsti.at[c0 + b]], add=True)

    plsc.subcore_barrier()
    pltpu.sync_copy(acc_sh.at[pl.ds(wb, DEG_WB)],
                    out_hbm.at[cid, pl.ds(wb, DEG_WB)])


@functools.partial(
    pl.kernel,
    out_type=jax.ShapeDtypeStruct((NUM_CORES, HALF, D), jnp.float32),
    mesh=_mesh,
    scratch_types=[
        pltpu.VMEM((AGG_ROWS_PER_TILE, 128), jnp.int32),
        pltpu.VMEM((AGG_ROWS_PER_TILE, 128), jnp.int32),
        pltpu.VMEM((128, D), jnp.float32),
        pltpu.VMEM((128, D), jnp.float32),
        pltpu.VMEM_SHARED((ACC_ROWS, D), jnp.float32),
        pltpu.SemaphoreType.DMA,
        pltpu.SemaphoreType.DMA,
        pltpu.SemaphoreType.DMA,
        pltpu.SemaphoreType.DMA,
    ],
)
def _sc_aggregate(hp_hbm, src_hbm, dstr_hbm, z_hbm, out_hbm,
                  srci, dsti, buf0, buf1, acc_sh,
                  g0, g1, s0, s1):
    bufs = (buf0, buf1)
    gsems = (g0, g1)
    ssems = (s0, s1)
    cid = lax.axis_index("c")
    sid = lax.axis_index("s")
    base = sid * AGG_ROWS_PER_TILE
    pltpu.sync_copy(src_hbm.at[pl.ds(base, AGG_ROWS_PER_TILE)], srci)
    pltpu.sync_copy(dstr_hbm.at[cid, pl.ds(base, AGG_ROWS_PER_TILE)], dsti)
    wb = sid * AGG_WB
    pltpu.sync_copy(z_hbm.at[pl.ds(wb, AGG_WB)], acc_sh.at[pl.ds(wb, AGG_WB)])
    plsc.subcore_barrier()

    for b in range(2):
        pltpu.async_copy(hp_hbm.at[srci.at[b]], bufs[b], gsems[b])

    @pl.loop(0, AGG_ROWS_PER_TILE, step=2)
    def _(c0):
        for b in range(2):
            j = c0 + b
            pltpu.make_async_copy(hp_hbm.at[srci.at[j]], bufs[b], gsems[b]).wait()
            pltpu.async_copy(bufs[b], acc_sh.at[dsti.at[j]], ssems[b], add=True)

            @pl.when(j + 2 < AGG_ROWS_PER_TILE)
            def _():
                pltpu.make_async_copy(bufs[b], acc_sh.at[dsti.at[j]],
                                      ssems[b]).wait()
                pltpu.async_copy(hp_hbm.at[srci.at[j + 2]], bufs[b], gsems[b])

    for b in range(2):
        pltpu.make_async_copy(bufs[b], acc_sh.at[dsti.at[b]], ssems[b]).wait()
    plsc.subcore_barrier()
    pltpu.sync_copy(acc_sh.at[pl.ds(wb, AGG_WB)],
                    out_hbm.at[cid, pl.ds(wb, AGG_WB)])


def _tc_matmul(xp, W):
    def body(x_ref, w_ref, o_ref):
        o_ref[...] = jnp.dot(x_ref[...], w_ref[...],
                             preferred_element_type=jnp.float32)

    return pl.pallas_call(
        body,
        grid=(GRID,),
        in_specs=[pl.BlockSpec((BLK, D), lambda i: (i, 0)),
                  pl.BlockSpec((D, D), lambda i: (0, 0))],
        out_specs=pl.BlockSpec((BLK, D), lambda i: (i, 0)),
        out_shape=jax.ShapeDtypeStruct((NPAD, D), jnp.float32),
    )(xp, W)


def _tc_scale(deg_parts, hm):
    def body(dp_ref, hm_ref, disb_ref, hp_ref):
        deg = dp_ref[0] + dp_ref[1]
        dis = 1.0 / jnp.sqrt(deg[:, 0:1] + 1.0)
        disb = jnp.broadcast_to(dis, (BLK, D))
        disb_ref[...] = disb
        hp_ref[...] = disb * hm_ref[...]

    return pl.pallas_call(
        body,
        grid=(GRID,),
        in_specs=[pl.BlockSpec((NUM_CORES, BLK, D), lambda i: (0, i, 0)),
                  pl.BlockSpec((BLK, D), lambda i: (i, 0))],
        out_specs=[pl.BlockSpec((BLK, D), lambda i: (i, 0)),
                   pl.BlockSpec((BLK, D), lambda i: (i, 0))],
        out_shape=[jax.ShapeDtypeStruct((NPAD, D), jnp.float32),
                   jax.ShapeDtypeStruct((NPAD, D), jnp.float32)],
    )(deg_parts, hm)


_ACC_SPEC = pl.BlockSpec((1, BLK, D), lambda i: (i // 4, i % 4, 0))


def _tc_layer(acc_parts, hp_prev, disb, bias, Wn):
    def body(a_ref, hp_ref, d_ref, b_ref, w_ref, o_ref):
        t = d_ref[...] * (a_ref[0] + hp_ref[...]) + b_ref[...]
        t = jnp.where(t >= 0, t, 0.01 * t)
        o_ref[...] = d_ref[...] * jnp.dot(t, w_ref[...],
                                          preferred_element_type=jnp.float32)

    return pl.pallas_call(
        body,
        grid=(GRID,),
        in_specs=[_ACC_SPEC,
                  pl.BlockSpec((BLK, D), lambda i: (i, 0)),
                  pl.BlockSpec((BLK, D), lambda i: (i, 0)),
                  pl.BlockSpec((1, D), lambda i: (0, 0)),
                  pl.BlockSpec((D, D), lambda i: (0, 0))],
        out_specs=pl.BlockSpec((BLK, D), lambda i: (i, 0)),
        out_shape=jax.ShapeDtypeStruct((NPAD, D), jnp.float32),
    )(acc_parts, hp_prev, disb, bias, Wn)


def _tc_final(acc_parts, hp_prev, disb, bias):
    def body(a_ref, hp_ref, d_ref, b_ref, o_ref):
        o_ref[...] = d_ref[...] * (a_ref[0] + hp_ref[...]) + b_ref[...]

    return pl.pallas_call(
        body,
        grid=(GRID,),
        in_specs=[_ACC_SPEC,
                  pl.BlockSpec((BLK, D), lambda i: (i, 0)),
                  pl.BlockSpec((BLK, D), lambda i: (i, 0)),
                  pl.BlockSpec((1, D), lambda i: (0, 0))],
        out_specs=pl.BlockSpec((BLK, D), lambda i: (i, 0)),
        out_shape=jax.ShapeDtypeStruct((NPAD, D), jnp.float32),
    )(acc_parts, hp_prev, disb, bias)


def kernel(x, edge_index, W1, b1, W2, b2, W3, b3):
    ei = edge_index.astype(jnp.int32)
    pad = jnp.full((EPAD - N_EDGES,), N_NODES, jnp.int32)
    src2d = jnp.concatenate([ei[0], pad]).reshape(EROWS, 128)
    dst = jnp.concatenate([ei[1], pad])
    dst_lo = jnp.where(dst < HALF, dst, TRASH).reshape(EROWS, 128)
    dst_hi = jnp.where(dst >= HALF, dst - HALF, TRASH).reshape(EROWS, 128)
    dstr = jnp.stack([dst_lo, dst_hi])
    dst2d = dst.reshape(EROWS, 128)
    xp = jnp.zeros((NPAD, D), jnp.float32).at[:N_NODES].set(x)
    zeros_acc = jnp.zeros((HALF, D), jnp.float32)
    zeros_deg = jnp.zeros((HALF, D), jnp.float32)
    ones16 = jnp.ones((128, D), jnp.float32)

    deg_parts = _sc_degree(dst2d, zeros_deg, ones16)
    hm1 = _tc_matmul(xp, W1)
    disb, hp1 = _tc_scale(deg_parts, hm1)

    acc1 = _sc_aggregate(hp1, src2d, dstr, zeros_acc)
    hp2 = _tc_layer(acc1, hp1, disb, b1.reshape(1, D), W2)
    acc2 = _sc_aggregate(hp2, src2d, dstr, zeros_acc)
    hp3 = _tc_layer(acc2, hp2, disb, b2.reshape(1, D), W3)
    acc3 = _sc_aggregate(hp3, src2d, dstr, zeros_acc)
    out = _tc_final(acc3, hp3, disb, b3.reshape(1, D))
    return out[:N_NODES]


# speedup vs baseline: 13.1134x; 2.7145x over previous
import functools

import jax
import jax.numpy as jnp
from jax import lax
from jax.experimental import pallas as pl
from jax.experimental.pallas import tpu as pltpu
from jax.experimental.pallas import tpu_sc as plsc

N_NODES = 10000
D = 128
N_EDGES = 320000

NUM_CORES = 2
NUM_SUBCORES = 16

NPAD = 10240
EROWS = 2560
EPAD = EROWS * 128
HALF = NPAD // 2
ACC_ROWS = HALF + 8
TRASH = HALF

DEG_ROWS_PER_TILE = EROWS // (NUM_CORES * NUM_SUBCORES)
AGG_CHUNK = 128
AGG_EROWS = EPAD // AGG_CHUNK
AGG_ROWS_PER_TILE = AGG_EROWS // NUM_SUBCORES
AGG_NBUF = 2
DEG_WB = NPAD // NUM_SUBCORES
AGG_WB = HALF // NUM_SUBCORES

BLK = 1280
GRID = NPAD // BLK

_mesh = plsc.VectorSubcoreMesh(core_axis_name="c", subcore_axis_name="s")


@functools.partial(
    pl.kernel,
    out_type=jax.ShapeDtypeStruct((NUM_CORES, NPAD, D), jnp.float32),
    mesh=_mesh,
    scratch_types=[
        pltpu.VMEM((DEG_ROWS_PER_TILE, 128), jnp.int32),
        pltpu.VMEM((128, D), jnp.float32),
        pltpu.VMEM_SHARED((NPAD, D), jnp.float32),
        pltpu.SemaphoreType.DMA,
    ],
)
def _sc_degree(dst_hbm, z_hbm, ones_hbm, out_hbm, dsti, ones_v, acc_sh, sem):
    cid = lax.axis_index("c")
    sid = lax.axis_index("s")
    base = (cid * NUM_SUBCORES + sid) * DEG_ROWS_PER_TILE
    pltpu.sync_copy(dst_hbm.at[pl.ds(base, DEG_ROWS_PER_TILE)], dsti)
    pltpu.sync_copy(ones_hbm, ones_v)
    wb = sid * DEG_WB
    pltpu.sync_copy(z_hbm.at[pl.ds(0, DEG_WB)], acc_sh.at[pl.ds(wb, DEG_WB)])
    plsc.subcore_barrier()

    @pl.loop(0, DEG_ROWS_PER_TILE, step=8)
    def _(c0):
        for b in range(8):
            pltpu.sync_copy(ones_v, acc_sh.at[dsti.at[c0 + b]], add=True)

    plsc.subcore_barrier()
    pltpu.sync_copy(acc_sh.at[pl.ds(wb, DEG_WB)],
                    out_hbm.at[cid, pl.ds(wb, DEG_WB)])


@functools.partial(
    pl.kernel,
    out_type=jax.ShapeDtypeStruct((NUM_CORES, HALF, D), jnp.float32),
    mesh=_mesh,
    scratch_types=[
        pltpu.VMEM((AGG_ROWS_PER_TILE, AGG_CHUNK), jnp.int32),
        pltpu.VMEM((AGG_ROWS_PER_TILE, AGG_CHUNK), jnp.int32),
        pltpu.VMEM((AGG_CHUNK, D), jnp.float32),
        pltpu.VMEM((AGG_CHUNK, D), jnp.float32),
        pltpu.VMEM_SHARED((ACC_ROWS, D), jnp.float32),
        pltpu.SemaphoreType.DMA,
        pltpu.SemaphoreType.DMA,
        pltpu.SemaphoreType.DMA,
        pltpu.SemaphoreType.DMA,
    ],
)
def _sc_aggregate(hp_hbm, src_hbm, dstr_hbm, z_hbm, out_hbm,
                  srci, dsti, buf0, buf1, acc_sh,
                  g0, g1, s0, s1):
    bufs = (buf0, buf1)
    gsems = (g0, g1)
    ssems = (s0, s1)
    cid = lax.axis_index("c")
    sid = lax.axis_index("s")
    base = sid * AGG_ROWS_PER_TILE
    pltpu.sync_copy(src_hbm.at[pl.ds(base, AGG_ROWS_PER_TILE)], srci)
    pltpu.sync_copy(dstr_hbm.at[cid, pl.ds(base, AGG_ROWS_PER_TILE)], dsti)
    wb = sid * AGG_WB
    pltpu.sync_copy(z_hbm.at[pl.ds(wb, AGG_WB)], acc_sh.at[pl.ds(wb, AGG_WB)])
    plsc.subcore_barrier()

    for b in range(AGG_NBUF):
        pltpu.async_copy(hp_hbm.at[srci.at[b]], bufs[b], gsems[b])

    @pl.loop(0, AGG_ROWS_PER_TILE, step=AGG_NBUF)
    def _(c0):
        for b in range(AGG_NBUF):
            j = c0 + b
            pltpu.make_async_copy(hp_hbm.at[srci.at[j]], bufs[b], gsems[b]).wait()
            pltpu.async_copy(bufs[b], acc_sh.at[dsti.at[j]], ssems[b], add=True)

            @pl.when(j + AGG_NBUF < AGG_ROWS_PER_TILE)
            def _():
                pltpu.make_async_copy(bufs[b], acc_sh.at[dsti.at[j]],
                                      ssems[b]).wait()
                pltpu.async_copy(hp_hbm.at[srci.at[j + AGG_NBUF]], bufs[b],
                                 gsems[b])

    for b in range(AGG_NBUF):
        pltpu.make_async_copy(bufs[b], acc_sh.at[dsti.at[b]], ssems[b]).wait()
    plsc.subcore_barrier()
    pltpu.sync_copy(acc_sh.at[pl.ds(wb, AGG_WB)],
                    out_hbm.at[cid, pl.ds(wb, AGG_WB)])


def _tc_matmul(xp, W):
    def body(x_ref, w_ref, o_ref):
        o_ref[...] = jnp.dot(x_ref[...], w_ref[...],
                             preferred_element_type=jnp.float32)

    return pl.pallas_call(
        body,
        grid=(GRID,),
        in_specs=[pl.BlockSpec((BLK, D), lambda i: (i, 0)),
                  pl.BlockSpec((D, D), lambda i: (0, 0))],
        out_specs=pl.BlockSpec((BLK, D), lambda i: (i, 0)),
        out_shape=jax.ShapeDtypeStruct((NPAD, D), jnp.float32),
    )(xp, W)


def _tc_scale(deg_parts, hm):
    def body(dp_ref, hm_ref, disb_ref, hp_ref):
        deg = dp_ref[0] + dp_ref[1]
        dis = 1.0 / jnp.sqrt(deg[:, 0:1] + 1.0)
        disb = jnp.broadcast_to(dis, (BLK, D))
        disb_ref[...] = disb
        hp_ref[...] = disb * hm_ref[...]

    return pl.pallas_call(
        body,
        grid=(GRID,),
        in_specs=[pl.BlockSpec((NUM_CORES, BLK, D), lambda i: (0, i, 0)),
                  pl.BlockSpec((BLK, D), lambda i: (i, 0))],
        out_specs=[pl.BlockSpec((BLK, D), lambda i: (i, 0)),
                   pl.BlockSpec((BLK, D), lambda i: (i, 0))],
        out_shape=[jax.ShapeDtypeStruct((NPAD, D), jnp.float32),
                   jax.ShapeDtypeStruct((NPAD, D), jnp.float32)],
    )(deg_parts, hm)


_ACC_SPEC = pl.BlockSpec((1, BLK, D), lambda i: (i // 4, i % 4, 0))


def _tc_layer(acc_parts, hp_prev, disb, bias, Wn):
    def body(a_ref, hp_ref, d_ref, b_ref, w_ref, o_ref):
        t = d_ref[...] * (a_ref[0] + hp_ref[...]) + b_ref[...]
        t = jnp.where(t >= 0, t, 0.01 * t)
        o_ref[...] = d_ref[...] * jnp.dot(t, w_ref[...],
                                          preferred_element_type=jnp.float32)

    return pl.pallas_call(
        body,
        grid=(GRID,),
        in_specs=[_ACC_SPEC,
                  pl.BlockSpec((BLK, D), lambda i: (i, 0)),
                  pl.BlockSpec((BLK, D), lambda i: (i, 0)),
                  pl.BlockSpec((1, D), lambda i: (0, 0)),
                  pl.BlockSpec((D, D), lambda i: (0, 0))],
        out_specs=pl.BlockSpec((BLK, D), lambda i: (i, 0)),
        out_shape=jax.ShapeDtypeStruct((NPAD, D), jnp.float32),
    )(acc_parts, hp_prev, disb, bias, Wn)


def _tc_final(acc_parts, hp_prev, disb, bias):
    def body(a_ref, hp_ref, d_ref, b_ref, o_ref):
        o_ref[...] = d_ref[...] * (a_ref[0] + hp_ref[...]) + b_ref[...]

    return pl.pallas_call(
        body,
        grid=(GRID,),
        in_specs=[_ACC_SPEC,
                  pl.BlockSpec((BLK, D), lambda i: (i, 0)),
                  pl.BlockSpec((BLK, D), lambda i: (i, 0)),
                  pl.BlockSpec((1, D), lambda i: (0, 0))],
        out_specs=pl.BlockSpec((BLK, D), lambda i: (i, 0)),
        out_shape=jax.ShapeDtypeStruct((NPAD, D), jnp.float32),
    )(acc_parts, hp_prev, disb, bias)


def kernel(x, edge_index, W1, b1, W2, b2, W3, b3):
    ei = edge_index.astype(jnp.int32)
    pad_src = jnp.arange(EPAD - N_EDGES, dtype=jnp.int32) % NPAD
    pad_dst = jnp.full((EPAD - N_EDGES,), N_NODES, jnp.int32)
    src2d = jnp.concatenate([ei[0], pad_src]).reshape(AGG_EROWS, AGG_CHUNK)
    dst = jnp.concatenate([ei[1], pad_dst])
    dst_lo = jnp.where(dst < HALF, dst, TRASH).reshape(AGG_EROWS, AGG_CHUNK)
    dst_hi = jnp.where(dst >= HALF, dst - HALF, TRASH).reshape(AGG_EROWS, AGG_CHUNK)
    dstr = jnp.stack([dst_lo, dst_hi])
    dst2d = dst.reshape(EROWS, 128)
    xp = jnp.zeros((NPAD, D), jnp.float32).at[:N_NODES].set(x)
    zeros_acc = jnp.zeros((HALF, D), jnp.float32)
    zeros_deg = jnp.zeros((HALF, D), jnp.float32)
    ones16 = jnp.ones((128, D), jnp.float32)

    deg_parts = _sc_degree(dst2d, zeros_deg, ones16)
    hm1 = _tc_matmul(xp, W1)
    disb, hp1 = _tc_scale(deg_parts, hm1)

    acc1 = _sc_aggregate(hp1, src2d, dstr, zeros_acc)
    hp2 = _tc_layer(acc1, hp1, disb, b1.reshape(1, D), W2)
    acc2 = _sc_aggregate(hp2, src2d, dstr, zeros_acc)
    hp3 = _tc_layer(acc2, hp2, disb, b2.reshape(1, D), W3)
    acc3 = _sc_aggregate(hp3, src2d, dstr, zeros_acc)
    out = _tc_final(acc3, hp3, disb, b3.reshape(1, D))
    return out[:N_NODES]


# speedup vs baseline: 13.1167x; 1.0003x over previous
import functools

import jax
import jax.numpy as jnp
from jax import lax
from jax.experimental import pallas as pl
from jax.experimental.pallas import tpu as pltpu
from jax.experimental.pallas import tpu_sc as plsc

N_NODES = 10000
D = 128
N_EDGES = 320000

NUM_CORES = 2
NUM_SUBCORES = 16

NPAD = 10240
EROWS = 2560
EPAD = EROWS * 128
HALF = NPAD // 2
ACC_ROWS = HALF + 8
TRASH = HALF

DEG_ROWS_PER_TILE = EROWS // (NUM_CORES * NUM_SUBCORES)
AGG_CHUNK = 128
AGG_EROWS = EPAD // AGG_CHUNK
AGG_ROWS_PER_TILE = AGG_EROWS // NUM_SUBCORES
AGG_NBUF = 2
DEG_WB = NPAD // NUM_SUBCORES
AGG_WB = HALF // NUM_SUBCORES

BLK = 1280
GRID = NPAD // BLK

_mesh = plsc.VectorSubcoreMesh(core_axis_name="c", subcore_axis_name="s")


@functools.partial(
    pl.kernel,
    out_type=jax.ShapeDtypeStruct((NUM_CORES, NPAD, D), jnp.float32),
    mesh=_mesh,
    scratch_types=[
        pltpu.VMEM((DEG_ROWS_PER_TILE, 128), jnp.int32),
        pltpu.VMEM((128, D), jnp.float32),
        pltpu.VMEM_SHARED((NPAD, D), jnp.float32),
        pltpu.SemaphoreType.DMA,
        pltpu.SemaphoreType.DMA,
    ],
)
def _sc_degree(dst_hbm, z_hbm, ones_hbm, out_hbm, dsti, ones_v, acc_sh, sem,
               sem2):
    cid = lax.axis_index("c")
    sid = lax.axis_index("s")
    base = (cid * NUM_SUBCORES + sid) * DEG_ROWS_PER_TILE
    pltpu.sync_copy(dst_hbm.at[pl.ds(base, DEG_ROWS_PER_TILE)], dsti)
    pltpu.sync_copy(ones_hbm, ones_v)
    wb = sid * DEG_WB
    pltpu.sync_copy(z_hbm.at[pl.ds(0, DEG_WB)], acc_sh.at[pl.ds(wb, DEG_WB)])
    plsc.subcore_barrier()

    sems = (sem, sem2)
    for b in range(2):
        pltpu.async_copy(ones_v, acc_sh.at[dsti.at[b]], sems[b], add=True)

    @pl.loop(2, DEG_ROWS_PER_TILE, step=2)
    def _(c0):
        for b in range(2):
            j = c0 + b
            pltpu.make_async_copy(ones_v, acc_sh.at[dsti.at[j - 2]],
                                  sems[b]).wait()
            pltpu.async_copy(ones_v, acc_sh.at[dsti.at[j]], sems[b], add=True)

    for b in range(2):
        j = DEG_ROWS_PER_TILE - 2 + b
        pltpu.make_async_copy(ones_v, acc_sh.at[dsti.at[j]], sems[b]).wait()
    plsc.subcore_barrier()
    pltpu.sync_copy(acc_sh.at[pl.ds(wb, DEG_WB)],
                    out_hbm.at[cid, pl.ds(wb, DEG_WB)])


@functools.partial(
    pl.kernel,
    out_type=jax.ShapeDtypeStruct((NUM_CORES, HALF, D), jnp.float32),
    mesh=_mesh,
    scratch_types=[
        pltpu.VMEM((AGG_ROWS_PER_TILE, AGG_CHUNK), jnp.int32),
        pltpu.VMEM((AGG_ROWS_PER_TILE, AGG_CHUNK), jnp.int32),
        pltpu.VMEM((AGG_CHUNK, D), jnp.float32),
        pltpu.VMEM((AGG_CHUNK, D), jnp.float32),
        pltpu.VMEM_SHARED((ACC_ROWS, D), jnp.float32),
        pltpu.SemaphoreType.DMA,
        pltpu.SemaphoreType.DMA,
        pltpu.SemaphoreType.DMA,
        pltpu.SemaphoreType.DMA,
    ],
)
def _sc_aggregate(hp_hbm, src_hbm, dstr_hbm, z_hbm, out_hbm,
                  srci, dsti, buf0, buf1, acc_sh,
                  g0, g1, s0, s1):
    bufs = (buf0, buf1)
    gsems = (g0, g1)
    ssems = (s0, s1)
    cid = lax.axis_index("c")
    sid = lax.axis_index("s")
    base = sid * AGG_ROWS_PER_TILE
    pltpu.sync_copy(src_hbm.at[pl.ds(base, AGG_ROWS_PER_TILE)], srci)
    pltpu.sync_copy(dstr_hbm.at[cid, pl.ds(base, AGG_ROWS_PER_TILE)], dsti)
    wb = sid * AGG_WB
    pltpu.sync_copy(z_hbm.at[pl.ds(wb, AGG_WB)], acc_sh.at[pl.ds(wb, AGG_WB)])
    plsc.subcore_barrier()

    for b in range(AGG_NBUF):
        pltpu.async_copy(hp_hbm.at[srci.at[b]], bufs[b], gsems[b])

    @pl.loop(0, AGG_ROWS_PER_TILE, step=AGG_NBUF)
    def _(c0):
        for b in range(AGG_NBUF):
            j = c0 + b

            @pl.when(jnp.logical_and(j > 0, j + 1 < AGG_ROWS_PER_TILE))
            def _():
                pltpu.make_async_copy(bufs[1 - b], acc_sh.at[dsti.at[j - 1]],
                                      ssems[1 - b]).wait()
                pltpu.async_copy(hp_hbm.at[srci.at[j + 1]], bufs[1 - b],
                                 gsems[1 - b])

            pltpu.make_async_copy(hp_hbm.at[srci.at[j]], bufs[b], gsems[b]).wait()
            pltpu.async_copy(bufs[b], acc_sh.at[dsti.at[j]], ssems[b], add=True)

    for b in range(AGG_NBUF):
        j = AGG_ROWS_PER_TILE - 2 + b
        pltpu.make_async_copy(bufs[j % 2], acc_sh.at[dsti.at[j]],
                              ssems[j % 2]).wait()
    plsc.subcore_barrier()
    pltpu.sync_copy(acc_sh.at[pl.ds(wb, AGG_WB)],
                    out_hbm.at[cid, pl.ds(wb, AGG_WB)])


def _tc_matmul(xp, W):
    def body(x_ref, w_ref, o_ref):
        o_ref[...] = jnp.dot(x_ref[...], w_ref[...],
                             preferred_element_type=jnp.float32)

    return pl.pallas_call(
        body,
        grid=(GRID,),
        in_specs=[pl.BlockSpec((BLK, D), lambda i: (i, 0)),
                  pl.BlockSpec((D, D), lambda i: (0, 0))],
        out_specs=pl.BlockSpec((BLK, D), lambda i: (i, 0)),
        out_shape=jax.ShapeDtypeStruct((NPAD, D), jnp.float32),
    )(xp, W)


def _tc_scale(deg_parts, hm):
    def body(dp_ref, hm_ref, disb_ref, hp_ref):
        deg = dp_ref[0] + dp_ref[1]
        dis = 1.0 / jnp.sqrt(deg[:, 0:1] + 1.0)
        disb = jnp.broadcast_to(dis, (BLK, D))
        disb_ref[...] = disb
        hp_ref[...] = disb * hm_ref[...]

    return pl.pallas_call(
        body,
        grid=(GRID,),
        in_specs=[pl.BlockSpec((NUM_CORES, BLK, D), lambda i: (0, i, 0)),
                  pl.BlockSpec((BLK, D), lambda i: (i, 0))],
        out_specs=[pl.BlockSpec((BLK, D), lambda i: (i, 0)),
                   pl.BlockSpec((BLK, D), lambda i: (i, 0))],
        out_shape=[jax.ShapeDtypeStruct((NPAD, D), jnp.float32),
                   jax.ShapeDtypeStruct((NPAD, D), jnp.float32)],
    )(deg_parts, hm)


_ACC_SPEC = pl.BlockSpec((1, BLK, D), lambda i: (i // 4, i % 4, 0))


def _tc_layer(acc_parts, hp_prev, disb, bias, Wn):
    def body(a_ref, hp_ref, d_ref, b_ref, w_ref, o_ref):
        t = d_ref[...] * (a_ref[0] + hp_ref[...]) + b_ref[...]
        t = jnp.where(t >= 0, t, 0.01 * t)
        o_ref[...] = d_ref[...] * jnp.dot(t, w_ref[...],
                                          preferred_element_type=jnp.float32)

    return pl.pallas_call(
        body,
        grid=(GRID,),
        in_specs=[_ACC_SPEC,
                  pl.BlockSpec((BLK, D), lambda i: (i, 0)),
                  pl.BlockSpec((BLK, D), lambda i: (i, 0)),
                  pl.BlockSpec((1, D), lambda i: (0, 0)),
                  pl.BlockSpec((D, D), lambda i: (0, 0))],
        out_specs=pl.BlockSpec((BLK, D), lambda i: (i, 0)),
        out_shape=jax.ShapeDtypeStruct((NPAD, D), jnp.float32),
    )(acc_parts, hp_prev, disb, bias, Wn)


def _tc_final(acc_parts, hp_prev, disb, bias):
    def body(a_ref, hp_ref, d_ref, b_ref, o_ref):
        o_ref[...] = d_ref[...] * (a_ref[0] + hp_ref[...]) + b_ref[...]

    return pl.pallas_call(
        body,
        grid=(GRID,),
        in_specs=[_ACC_SPEC,
                  pl.BlockSpec((BLK, D), lambda i: (i, 0)),
                  pl.BlockSpec((BLK, D), lambda i: (i, 0)),
                  pl.BlockSpec((1, D), lambda i: (0, 0))],
        out_specs=pl.BlockSpec((BLK, D), lambda i: (i, 0)),
        out_shape=jax.ShapeDtypeStruct((NPAD, D), jnp.float32),
    )(acc_parts, hp_prev, disb, bias)


def kernel(x, edge_index, W1, b1, W2, b2, W3, b3):
    ei = edge_index.astype(jnp.int32)
    pad_src = jnp.arange(EPAD - N_EDGES, dtype=jnp.int32) % NPAD
    pad_dst = jnp.full((EPAD - N_EDGES,), N_NODES, jnp.int32)
    src2d = jnp.concatenate([ei[0], pad_src]).reshape(AGG_EROWS, AGG_CHUNK)
    dst = jnp.concatenate([ei[1], pad_dst])
    dst_lo = jnp.where(dst < HALF, dst, TRASH).reshape(AGG_EROWS, AGG_CHUNK)
    dst_hi = jnp.where(dst >= HALF, dst - HALF, TRASH).reshape(AGG_EROWS, AGG_CHUNK)
    dstr = jnp.stack([dst_lo, dst_hi])
    dst2d = dst.reshape(EROWS, 128)
    xp = jnp.zeros((NPAD, D), jnp.float32).at[:N_NODES].set(x)
    zeros_acc = jnp.zeros((HALF, D), jnp.float32)
    zeros_deg = jnp.zeros((HALF, D), jnp.float32)
    ones16 = jnp.ones((128, D), jnp.float32)

    deg_parts = _sc_degree(dst2d, zeros_deg, ones16)
    hm1 = _tc_matmul(xp, W1)
    disb, hp1 = _tc_scale(deg_parts, hm1)

    acc1 = _sc_aggregate(hp1, src2d, dstr, zeros_acc)
    hp2 = _tc_layer(acc1, hp1, disb, b1.reshape(1, D), W2)
    acc2 = _sc_aggregate(hp2, src2d, dstr, zeros_acc)
    hp3 = _tc_layer(acc2, hp2, disb, b2.reshape(1, D), W3)
    acc3 = _sc_aggregate(hp3, src2d, dstr, zeros_acc)
    out = _tc_final(acc3, hp3, disb, b3.reshape(1, D))
    return out[:N_NODES]


# speedup vs baseline: 21.3799x; 1.6300x over previous
import functools

import jax
import jax.numpy as jnp
from jax import lax
from jax.experimental import pallas as pl
from jax.experimental.pallas import tpu as pltpu
from jax.experimental.pallas import tpu_sc as plsc

N_NODES = 10000
D = 128
N_EDGES = 320000

NUM_CORES = 2
NUM_SUBCORES = 16

NPAD = 10240
EROWS = 2560
EPAD = EROWS * 128
HALF = NPAD // 2
ACC_ROWS = HALF + 8
TRASH = HALF

DEG_ROWS_PER_TILE = EROWS // (NUM_CORES * NUM_SUBCORES)
SEG = EPAD // (NUM_CORES * NUM_SUBCORES)
SEG_ROWS = SEG // 128
SEG_CHUNKS = SEG // 16
AGG_CHUNK = 128
AGG_EROWS = EPAD // AGG_CHUNK
AGG_ROWS_PER_TILE = AGG_EROWS // NUM_SUBCORES
AGG_NBUF = 2
DEG_WB = NPAD // NUM_SUBCORES
AGG_WB = HALF // NUM_SUBCORES

BLK = 1280
GRID = NPAD // BLK

_mesh = plsc.VectorSubcoreMesh(core_axis_name="c", subcore_axis_name="s")


@functools.partial(
    pl.kernel,
    out_type=jax.ShapeDtypeStruct((NUM_CORES, NPAD, D), jnp.float32),
    mesh=_mesh,
    scratch_types=[
        pltpu.VMEM((DEG_ROWS_PER_TILE, 128), jnp.int32),
        pltpu.VMEM((128, D), jnp.float32),
        pltpu.VMEM_SHARED((NPAD, D), jnp.float32),
        pltpu.SemaphoreType.DMA,
        pltpu.SemaphoreType.DMA,
    ],
)
def _sc_degree(dst_hbm, z_hbm, ones_hbm, out_hbm, dsti, ones_v, acc_sh, sem,
               sem2):
    cid = lax.axis_index("c")
    sid = lax.axis_index("s")
    base = (cid * NUM_SUBCORES + sid) * DEG_ROWS_PER_TILE
    pltpu.sync_copy(dst_hbm.at[pl.ds(base, DEG_ROWS_PER_TILE)], dsti)
    pltpu.sync_copy(ones_hbm, ones_v)
    wb = sid * DEG_WB
    pltpu.sync_copy(z_hbm.at[pl.ds(0, DEG_WB)], acc_sh.at[pl.ds(wb, DEG_WB)])
    plsc.subcore_barrier()

    sems = (sem, sem2)
    for b in range(2):
        pltpu.async_copy(ones_v, acc_sh.at[dsti.at[b]], sems[b], add=True)

    @pl.loop(2, DEG_ROWS_PER_TILE, step=2)
    def _(c0):
        for b in range(2):
            j = c0 + b
            pltpu.make_async_copy(ones_v, acc_sh.at[dsti.at[j - 2]],
                                  sems[b]).wait()
            pltpu.async_copy(ones_v, acc_sh.at[dsti.at[j]], sems[b], add=True)

    for b in range(2):
        j = DEG_ROWS_PER_TILE - 2 + b
        pltpu.make_async_copy(ones_v, acc_sh.at[dsti.at[j]], sems[b]).wait()
    plsc.subcore_barrier()
    pltpu.sync_copy(acc_sh.at[pl.ds(wb, DEG_WB)],
                    out_hbm.at[cid, pl.ds(wb, DEG_WB)])


@functools.partial(
    pl.kernel,
    out_type=[jax.ShapeDtypeStruct((NUM_CORES, 32, SEG), jnp.int32),
              jax.ShapeDtypeStruct((NUM_CORES, 32, SEG), jnp.int32),
              jax.ShapeDtypeStruct((NUM_CORES, 32, 16), jnp.int32)],
    mesh=_mesh,
    scratch_types=[
        pltpu.VMEM((SEG,), jnp.int32),
        pltpu.VMEM((SEG,), jnp.int32),
        pltpu.VMEM((SEG,), jnp.int32),
        pltpu.VMEM((SEG,), jnp.int32),
        pltpu.VMEM((SEG,), jnp.int32),
        pltpu.VMEM((SEG,), jnp.int32),
        pltpu.VMEM((16,), jnp.int32),
    ],
    compiler_params=pltpu.CompilerParams(needs_layout_passes=False),
)
def _sc_partition(srcf_hbm, dstf_hbm, fsrc_hbm, fdst_hbm,
                  osrc_hbm, odst_hbm, ocnt_hbm,
                  srcv, dstv, cs0, cd0, cs1, cd1, cntv):
    cid = lax.axis_index("c")
    sid = lax.axis_index("s")
    t = cid * NUM_SUBCORES + sid
    pltpu.sync_copy(srcf_hbm.at[pl.ds(t * SEG, SEG)], srcv)
    pltpu.sync_copy(dstf_hbm.at[pl.ds(t * SEG, SEG)], dstv)
    pltpu.sync_copy(fsrc_hbm, cs0)
    pltpu.sync_copy(fsrc_hbm, cs1)
    pltpu.sync_copy(fdst_hbm, cd0)
    pltpu.sync_copy(fdst_hbm, cd1)

    def chunk(i, carry):
        o0, o1 = carry
        vs = srcv[pl.ds(i * 16, 16)]
        vd = dstv[pl.ds(i * 16, 16)]
        m0 = vd < HALF
        plsc.store_compressed(cs0.at[pl.ds(o0, 16)], vs, mask=m0)
        plsc.store_compressed(cd0.at[pl.ds(o0, 16)], vd, mask=m0)
        m1 = jnp.logical_not(m0)
        plsc.store_compressed(cs1.at[pl.ds(o1, 16)], vs, mask=m1)
        plsc.store_compressed(cd1.at[pl.ds(o1, 16)], vd - HALF, mask=m1)
        n0 = jnp.sum(m0.astype(jnp.int32))
        return (o0 + n0, o1 + (16 - n0))

    o0, o1 = lax.fori_loop(0, SEG_CHUNKS, chunk, (0, 0))
    pltpu.sync_copy(cs0, osrc_hbm.at[0, t])
    pltpu.sync_copy(cd0, odst_hbm.at[0, t])
    pltpu.sync_copy(cs1, osrc_hbm.at[1, t])
    pltpu.sync_copy(cd1, odst_hbm.at[1, t])
    cntv[...] = jnp.full((16,), o0, jnp.int32)
    pltpu.sync_copy(cntv, ocnt_hbm.at[0, t])
    cntv[...] = jnp.full((16,), o1, jnp.int32)
    pltpu.sync_copy(cntv, ocnt_hbm.at[1, t])


@functools.partial(
    pl.kernel,
    out_type=jax.ShapeDtypeStruct((NUM_CORES, HALF, D), jnp.float32),
    mesh=_mesh,
    scratch_types=[
        pltpu.VMEM((SEG_ROWS, 128), jnp.int32),
        pltpu.VMEM((SEG_ROWS, 128), jnp.int32),
        pltpu.VMEM((SEG_ROWS, 128), jnp.int32),
        pltpu.VMEM((SEG_ROWS, 128), jnp.int32),
        pltpu.VMEM((16,), jnp.int32),
        pltpu.VMEM((AGG_CHUNK, D), jnp.float32),
        pltpu.VMEM((AGG_CHUNK, D), jnp.float32),
        pltpu.VMEM_SHARED((ACC_ROWS, D), jnp.float32),
        pltpu.SemaphoreType.DMA,
        pltpu.SemaphoreType.DMA,
        pltpu.SemaphoreType.DMA,
        pltpu.SemaphoreType.DMA,
    ],
    compiler_params=pltpu.CompilerParams(needs_layout_passes=False),
)
def _sc_aggregate(hp_hbm, seg_src_hbm, seg_dst_hbm, cnt_hbm, z_hbm, out_hbm,
                  srci0, dsti0, srci1, dsti1, cntv, buf0, buf1, acc_sh,
                  g0, g1, s0, s1):
    bufs = (buf0, buf1)
    gsems = (g0, g1)
    ssems = (s0, s1)
    cid = lax.axis_index("c")
    sid = lax.axis_index("s")
    t0 = 2 * sid
    pltpu.sync_copy(seg_src_hbm.at[cid, t0], srci0)
    pltpu.sync_copy(seg_dst_hbm.at[cid, t0], dsti0)
    pltpu.sync_copy(seg_src_hbm.at[cid, t0 + 1], srci1)
    pltpu.sync_copy(seg_dst_hbm.at[cid, t0 + 1], dsti1)
    pltpu.sync_copy(cnt_hbm.at[cid, t0], cntv)
    n0 = jnp.max(cntv[...])
    pltpu.sync_copy(cnt_hbm.at[cid, t0 + 1], cntv)
    n1 = jnp.max(cntv[...])
    wb = sid * AGG_WB
    pltpu.sync_copy(z_hbm.at[pl.ds(wb, AGG_WB)], acc_sh.at[pl.ds(wb, AGG_WB)])
    plsc.subcore_barrier()

    def run_segment(srci, dsti, n):
        rows = jnp.minimum(jnp.maximum(((n + 255) // 256) * 2, 2), SEG_ROWS)
        for b in range(2):
            pltpu.async_copy(hp_hbm.at[srci.at[b]], bufs[b], gsems[b])

        def step(i, _):
            c0 = 2 * i
            for b in range(2):
                j = c0 + b
                pltpu.make_async_copy(hp_hbm.at[srci.at[j]], bufs[b],
                                      gsems[b]).wait()
                pltpu.async_copy(bufs[b], acc_sh.at[dsti.at[j]], ssems[b],
                                 add=True)

                @pl.when(j + 2 < rows)
                def _():
                    pltpu.make_async_copy(bufs[b], acc_sh.at[dsti.at[j]],
                                          ssems[b]).wait()
                    pltpu.async_copy(hp_hbm.at[srci.at[j + 2]], bufs[b],
                                     gsems[b])
            return 0

        lax.fori_loop(0, rows // 2, step, 0)
        for b in range(2):
            pltpu.make_async_copy(bufs[b], acc_sh.at[dsti.at[b]],
                                  ssems[b]).wait()

    run_segment(srci0, dsti0, n0)
    run_segment(srci1, dsti1, n1)
    plsc.subcore_barrier()
    pltpu.sync_copy(acc_sh.at[pl.ds(wb, AGG_WB)],
                    out_hbm.at[cid, pl.ds(wb, AGG_WB)])


def _tc_matmul(xp, W):
    def body(x_ref, w_ref, o_ref):
        o_ref[...] = jnp.dot(x_ref[...], w_ref[...],
                             preferred_element_type=jnp.float32)

    return pl.pallas_call(
        body,
        grid=(GRID,),
        in_specs=[pl.BlockSpec((BLK, D), lambda i: (i, 0)),
                  pl.BlockSpec((D, D), lambda i: (0, 0))],
        out_specs=pl.BlockSpec((BLK, D), lambda i: (i, 0)),
        out_shape=jax.ShapeDtypeStruct((NPAD, D), jnp.float32),
    )(xp, W)


def _tc_scale(deg_parts, hm):
    def body(dp_ref, hm_ref, disb_ref, hp_ref):
        deg = dp_ref[0] + dp_ref[1]
        dis = 1.0 / jnp.sqrt(deg[:, 0:1] + 1.0)
        disb = jnp.broadcast_to(dis, (BLK, D))
        disb_ref[...] = disb
        hp_ref[...] = disb * hm_ref[...]

    return pl.pallas_call(
        body,
        grid=(GRID,),
        in_specs=[pl.BlockSpec((NUM_CORES, BLK, D), lambda i: (0, i, 0)),
                  pl.BlockSpec((BLK, D), lambda i: (i, 0))],
        out_specs=[pl.BlockSpec((BLK, D), lambda i: (i, 0)),
                   pl.BlockSpec((BLK, D), lambda i: (i, 0))],
        out_shape=[jax.ShapeDtypeStruct((NPAD, D), jnp.float32),
                   jax.ShapeDtypeStruct((NPAD, D), jnp.float32)],
    )(deg_parts, hm)


_ACC_SPEC = pl.BlockSpec((1, BLK, D), lambda i: (i // 4, i % 4, 0))


def _tc_layer(acc_parts, hp_prev, disb, bias, Wn):
    def body(a_ref, hp_ref, d_ref, b_ref, w_ref, o_ref):
        t = d_ref[...] * (a_ref[0] + hp_ref[...]) + b_ref[...]
        t = jnp.where(t >= 0, t, 0.01 * t)
        o_ref[...] = d_ref[...] * jnp.dot(t, w_ref[...],
                                          preferred_element_type=jnp.float32)

    return pl.pallas_call(
        body,
        grid=(GRID,),
        in_specs=[_ACC_SPEC,
                  pl.BlockSpec((BLK, D), lambda i: (i, 0)),
                  pl.BlockSpec((BLK, D), lambda i: (i, 0)),
                  pl.BlockSpec((1, D), lambda i: (0, 0)),
                  pl.BlockSpec((D, D), lambda i: (0, 0))],
        out_specs=pl.BlockSpec((BLK, D), lambda i: (i, 0)),
        out_shape=jax.ShapeDtypeStruct((NPAD, D), jnp.float32),
    )(acc_parts, hp_prev, disb, bias, Wn)


def _tc_final(acc_parts, hp_prev, disb, bias):
    def body(a_ref, hp_ref, d_ref, b_ref, o_ref):
        o_ref[...] = d_ref[...] * (a_ref[0] + hp_ref[...]) + b_ref[...]

    return pl.pallas_call(
        body,
        grid=(GRID,),
        in_specs=[_ACC_SPEC,
                  pl.BlockSpec((BLK, D), lambda i: (i, 0)),
                  pl.BlockSpec((BLK, D), lambda i: (i, 0)),
                  pl.BlockSpec((1, D), lambda i: (0, 0))],
        out_specs=pl.BlockSpec((BLK, D), lambda i: (i, 0)),
        out_shape=jax.ShapeDtypeStruct((NPAD, D), jnp.float32),
    )(acc_parts, hp_prev, disb, bias)


def kernel(x, edge_index, W1, b1, W2, b2, W3, b3):
    ei = edge_index.astype(jnp.int32)
    pad_src = jnp.arange(EPAD - N_EDGES, dtype=jnp.int32) % NPAD
    pad_dst = jnp.full((EPAD - N_EDGES,), N_NODES, jnp.int32)
    srcf = jnp.concatenate([ei[0], pad_src])
    dstf = jnp.concatenate([ei[1], pad_dst])
    dst2d = dstf.reshape(EROWS, 128)
    fill_src = jnp.arange(SEG, dtype=jnp.int32) % NPAD
    fill_dst = jnp.full((SEG,), TRASH, jnp.int32)
    xp = jnp.zeros((NPAD, D), jnp.float32).at[:N_NODES].set(x)
    zeros_acc = jnp.zeros((HALF, D), jnp.float32)
    zeros_deg = jnp.zeros((HALF, D), jnp.float32)
    ones16 = jnp.ones((128, D), jnp.float32)

    seg_src, seg_dst, cnt = _sc_partition(srcf, dstf, fill_src, fill_dst)
    seg_src = seg_src.reshape(NUM_CORES, 32, SEG_ROWS, 128)
    seg_dst = seg_dst.reshape(NUM_CORES, 32, SEG_ROWS, 128)
    deg_parts = _sc_degree(dst2d, zeros_deg, ones16)
    hm1 = _tc_matmul(xp, W1)
    disb, hp1 = _tc_scale(deg_parts, hm1)

    acc1 = _sc_aggregate(hp1, seg_src, seg_dst, cnt, zeros_acc)
    hp2 = _tc_layer(acc1, hp1, disb, b1.reshape(1, D), W2)
    acc2 = _sc_aggregate(hp2, seg_src, seg_dst, cnt, zeros_acc)
    hp3 = _tc_layer(acc2, hp2, disb, b2.reshape(1, D), W3)
    acc3 = _sc_aggregate(hp3, seg_src, seg_dst, cnt, zeros_acc)
    out = _tc_final(acc3, hp3, disb, b3.reshape(1, D))
    return out[:N_NODES]


# speedup vs baseline: 23.4822x; 1.0983x over previous
import functools

import jax
import jax.numpy as jnp
from jax import lax
from jax.experimental import pallas as pl
from jax.experimental.pallas import tpu as pltpu
from jax.experimental.pallas import tpu_sc as plsc

N_NODES = 10000
D = 128
N_EDGES = 320000

NUM_CORES = 2
NUM_SUBCORES = 16

NPAD = 10240
EROWS = 2560
EPAD = EROWS * 128
HALF = NPAD // 2
ACC_ROWS = HALF + 8
TRASH = HALF

DEG_ROWS_PER_TILE = EROWS // (NUM_CORES * NUM_SUBCORES)
SEG = EPAD // (NUM_CORES * NUM_SUBCORES)
SEG_ROWS = SEG // 128
SEG_CHUNKS = SEG // 16
AGG_CHUNK = 128
AGG_EROWS = EPAD // AGG_CHUNK
AGG_ROWS_PER_TILE = AGG_EROWS // NUM_SUBCORES
AGG_NBUF = 2
DEG_WB = NPAD // NUM_SUBCORES
AGG_WB = HALF // NUM_SUBCORES

BLK = 1280
GRID = NPAD // BLK

_mesh = plsc.VectorSubcoreMesh(core_axis_name="c", subcore_axis_name="s")


@functools.partial(
    pl.kernel,
    out_type=[jax.ShapeDtypeStruct((NUM_CORES, 32, SEG), jnp.int32),
              jax.ShapeDtypeStruct((NUM_CORES, 32, SEG), jnp.int32),
              jax.ShapeDtypeStruct((NUM_CORES, 32, 16), jnp.int32),
              jax.ShapeDtypeStruct((NUM_CORES, NPAD // 128, 128), jnp.float32)],
    mesh=_mesh,
    scratch_types=[
        pltpu.VMEM((SEG,), jnp.int32),
        pltpu.VMEM((SEG,), jnp.int32),
        pltpu.VMEM((SEG,), jnp.int32),
        pltpu.VMEM((SEG,), jnp.int32),
        pltpu.VMEM((SEG,), jnp.int32),
        pltpu.VMEM((SEG,), jnp.int32),
        pltpu.VMEM((16,), jnp.int32),
        pltpu.VMEM((NPAD // 128, 128), jnp.float32),
        pltpu.VMEM((1, NPAD // 128), jnp.int32),
        pltpu.VMEM_SHARED((NPAD // 128, 128), jnp.float32),
    ],
    compiler_params=pltpu.CompilerParams(needs_layout_passes=False),
)
def _sc_partition(srcf_hbm, dstf_hbm, fsrc_hbm, fdst_hbm, z2_hbm, iota_hbm,
                  osrc_hbm, odst_hbm, ocnt_hbm, odeg_hbm,
                  srcv, dstv, cs0, cd0, cs1, cd1, cntv, degp, idv, deg_sh):
    cid = lax.axis_index("c")
    sid = lax.axis_index("s")
    t = cid * NUM_SUBCORES + sid
    pltpu.sync_copy(srcf_hbm.at[pl.ds(t * SEG, SEG)], srcv)
    pltpu.sync_copy(dstf_hbm.at[pl.ds(t * SEG, SEG)], dstv)
    pltpu.sync_copy(fsrc_hbm, cs0)
    pltpu.sync_copy(fsrc_hbm, cs1)
    pltpu.sync_copy(fdst_hbm, cd0)
    pltpu.sync_copy(fdst_hbm, cd1)
    pltpu.sync_copy(z2_hbm, degp)
    pltpu.sync_copy(iota_hbm, idv)
    @pl.when(sid < 10)
    def _():
        pltpu.sync_copy(z2_hbm.at[pl.ds(sid * 8, 8)],
                        deg_sh.at[pl.ds(sid * 8, 8)])
    plsc.subcore_barrier()

    ones_f = jnp.ones((16,), jnp.float32)

    def chunk(i, carry):
        o0, o1 = carry
        vs = srcv[pl.ds(i * 16, 16)]
        vd = dstv[pl.ds(i * 16, 16)]
        plsc.addupdate_scatter(degp, [vd >> 7, vd & 127], ones_f)
        m0 = vd < HALF
        plsc.store_compressed(cs0.at[pl.ds(o0, 16)], vs, mask=m0)
        plsc.store_compressed(cd0.at[pl.ds(o0, 16)], vd, mask=m0)
        m1 = jnp.logical_not(m0)
        plsc.store_compressed(cs1.at[pl.ds(o1, 16)], vs, mask=m1)
        plsc.store_compressed(cd1.at[pl.ds(o1, 16)], vd - HALF, mask=m1)
        n0 = jnp.sum(m0.astype(jnp.int32))
        return (o0 + n0, o1 + (16 - n0))

    o0, o1 = lax.fori_loop(0, SEG_CHUNKS, chunk, (0, 0))
    pltpu.sync_copy(cs0, osrc_hbm.at[0, t])
    pltpu.sync_copy(cd0, odst_hbm.at[0, t])
    pltpu.sync_copy(cs1, osrc_hbm.at[1, t])
    pltpu.sync_copy(cd1, odst_hbm.at[1, t])
    cntv[...] = jnp.full((16,), o0, jnp.int32)
    pltpu.sync_copy(cntv, ocnt_hbm.at[0, t])
    cntv[...] = jnp.full((16,), o1, jnp.int32)
    pltpu.sync_copy(cntv, ocnt_hbm.at[1, t])
    pltpu.sync_copy(degp, deg_sh.at[idv.at[0]], add=True)
    plsc.subcore_barrier()

    @pl.when(sid < 10)
    def _():
        pltpu.sync_copy(deg_sh.at[pl.ds(sid * 8, 8)],
                        odeg_hbm.at[cid, pl.ds(sid * 8, 8)])


@functools.partial(
    pl.kernel,
    out_type=jax.ShapeDtypeStruct((NUM_CORES, HALF, D), jnp.float32),
    mesh=_mesh,
    scratch_types=[
        pltpu.VMEM((SEG_ROWS, 128), jnp.int32),
        pltpu.VMEM((SEG_ROWS, 128), jnp.int32),
        pltpu.VMEM((SEG_ROWS, 128), jnp.int32),
        pltpu.VMEM((SEG_ROWS, 128), jnp.int32),
        pltpu.VMEM((16,), jnp.int32),
        pltpu.VMEM((AGG_CHUNK, D), jnp.float32),
        pltpu.VMEM((AGG_CHUNK, D), jnp.float32),
        pltpu.VMEM_SHARED((ACC_ROWS, D), jnp.float32),
        pltpu.SemaphoreType.DMA,
        pltpu.SemaphoreType.DMA,
        pltpu.SemaphoreType.DMA,
        pltpu.SemaphoreType.DMA,
    ],
    compiler_params=pltpu.CompilerParams(needs_layout_passes=False),
)
def _sc_aggregate(hp_hbm, seg_src_hbm, seg_dst_hbm, cnt_hbm, z_hbm, out_hbm,
                  srci0, dsti0, srci1, dsti1, cntv, buf0, buf1, acc_sh,
                  g0, g1, s0, s1):
    bufs = (buf0, buf1)
    gsems = (g0, g1)
    ssems = (s0, s1)
    cid = lax.axis_index("c")
    sid = lax.axis_index("s")
    t0 = 2 * sid
    pltpu.sync_copy(seg_src_hbm.at[cid, t0], srci0)
    pltpu.sync_copy(seg_dst_hbm.at[cid, t0], dsti0)
    pltpu.sync_copy(seg_src_hbm.at[cid, t0 + 1], srci1)
    pltpu.sync_copy(seg_dst_hbm.at[cid, t0 + 1], dsti1)
    pltpu.sync_copy(cnt_hbm.at[cid, t0], cntv)
    n0 = jnp.max(cntv[...])
    pltpu.sync_copy(cnt_hbm.at[cid, t0 + 1], cntv)
    n1 = jnp.max(cntv[...])
    wb = sid * AGG_WB
    pltpu.sync_copy(z_hbm.at[pl.ds(wb, AGG_WB)], acc_sh.at[pl.ds(wb, AGG_WB)])
    plsc.subcore_barrier()

    def run_segment(srci, dsti, n):
        rows = jnp.minimum(jnp.maximum(((n + 255) // 256) * 2, 2), SEG_ROWS)
        for b in range(2):
            pltpu.async_copy(hp_hbm.at[srci.at[b]], bufs[b], gsems[b])

        def step(i, _):
            c0 = 2 * i
            for b in range(2):
                j = c0 + b
                pltpu.make_async_copy(hp_hbm.at[srci.at[j]], bufs[b],
                                      gsems[b]).wait()
                pltpu.async_copy(bufs[b], acc_sh.at[dsti.at[j]], ssems[b],
                                 add=True)

                @pl.when(j + 2 < rows)
                def _():
                    pltpu.make_async_copy(bufs[b], acc_sh.at[dsti.at[j]],
                                          ssems[b]).wait()
                    pltpu.async_copy(hp_hbm.at[srci.at[j + 2]], bufs[b],
                                     gsems[b])
            return 0

        lax.fori_loop(0, rows // 2, step, 0)
        for b in range(2):
            pltpu.make_async_copy(bufs[b], acc_sh.at[dsti.at[b]],
                                  ssems[b]).wait()

    run_segment(srci0, dsti0, n0)
    run_segment(srci1, dsti1, n1)
    plsc.subcore_barrier()
    pltpu.sync_copy(acc_sh.at[pl.ds(wb, AGG_WB)],
                    out_hbm.at[cid, pl.ds(wb, AGG_WB)])


def _tc_matmul(xp, W):
    def body(x_ref, w_ref, o_ref):
        o_ref[...] = jnp.dot(x_ref[...], w_ref[...],
                             preferred_element_type=jnp.float32)

    return pl.pallas_call(
        body,
        grid=(GRID,),
        in_specs=[pl.BlockSpec((BLK, D), lambda i: (i, 0)),
                  pl.BlockSpec((D, D), lambda i: (0, 0))],
        out_specs=pl.BlockSpec((BLK, D), lambda i: (i, 0)),
        out_shape=jax.ShapeDtypeStruct((NPAD, D), jnp.float32),
    )(xp, W)


def _tc_scale(deg_parts, hm):
    def body(dp_ref, hm_ref, disb_ref, hp_ref):
        deg = dp_ref[0] + dp_ref[1]
        dis = 1.0 / jnp.sqrt(deg[:, 0:1] + 1.0)
        disb = jnp.broadcast_to(dis, (BLK, D))
        disb_ref[...] = disb
        hp_ref[...] = disb * hm_ref[...]

    return pl.pallas_call(
        body,
        grid=(GRID,),
        in_specs=[pl.BlockSpec((NUM_CORES, BLK, 16), lambda i: (0, i, 0)),
                  pl.BlockSpec((BLK, D), lambda i: (i, 0))],
        out_specs=[pl.BlockSpec((BLK, D), lambda i: (i, 0)),
                   pl.BlockSpec((BLK, D), lambda i: (i, 0))],
        out_shape=[jax.ShapeDtypeStruct((NPAD, D), jnp.float32),
                   jax.ShapeDtypeStruct((NPAD, D), jnp.float32)],
    )(deg_parts, hm)


_ACC_SPEC = pl.BlockSpec((1, BLK, D), lambda i: (i // 4, i % 4, 0))


def _tc_layer(acc_parts, hp_prev, disb, bias, Wn):
    def body(a_ref, hp_ref, d_ref, b_ref, w_ref, o_ref):
        t = d_ref[...] * (a_ref[0] + hp_ref[...]) + b_ref[...]
        t = jnp.where(t >= 0, t, 0.01 * t)
        o_ref[...] = d_ref[...] * jnp.dot(t, w_ref[...],
                                          preferred_element_type=jnp.float32)

    return pl.pallas_call(
        body,
        grid=(GRID,),
        in_specs=[_ACC_SPEC,
                  pl.BlockSpec((BLK, D), lambda i: (i, 0)),
                  pl.BlockSpec((BLK, D), lambda i: (i, 0)),
                  pl.BlockSpec((1, D), lambda i: (0, 0)),
                  pl.BlockSpec((D, D), lambda i: (0, 0))],
        out_specs=pl.BlockSpec((BLK, D), lambda i: (i, 0)),
        out_shape=jax.ShapeDtypeStruct((NPAD, D), jnp.float32),
    )(acc_parts, hp_prev, disb, bias, Wn)


def _tc_final(acc_parts, hp_prev, disb, bias):
    def body(a_ref, hp_ref, d_ref, b_ref, o_ref):
        o_ref[...] = d_ref[...] * (a_ref[0] + hp_ref[...]) + b_ref[...]

    return pl.pallas_call(
        body,
        grid=(GRID,),
        in_specs=[_ACC_SPEC,
                  pl.BlockSpec((BLK, D), lambda i: (i, 0)),
                  pl.BlockSpec((BLK, D), lambda i: (i, 0)),
                  pl.BlockSpec((1, D), lambda i: (0, 0))],
        out_specs=pl.BlockSpec((BLK, D), lambda i: (i, 0)),
        out_shape=jax.ShapeDtypeStruct((NPAD, D), jnp.float32),
    )(acc_parts, hp_prev, disb, bias)


def kernel(x, edge_index, W1, b1, W2, b2, W3, b3):
    ei = edge_index.astype(jnp.int32)
    pad_src = jnp.arange(EPAD - N_EDGES, dtype=jnp.int32) % NPAD
    pad_dst = jnp.full((EPAD - N_EDGES,), N_NODES, jnp.int32)
    srcf = jnp.concatenate([ei[0], pad_src])
    dstf = jnp.concatenate([ei[1], pad_dst])
    fill_src = jnp.arange(SEG, dtype=jnp.int32) % NPAD
    fill_dst = jnp.full((SEG,), TRASH, jnp.int32)
    zeros_hist = jnp.zeros((NPAD // 128, 128), jnp.float32)
    iota_rows = jnp.arange(NPAD // 128, dtype=jnp.int32)[None, :]
    xp = jnp.zeros((NPAD, D), jnp.float32).at[:N_NODES].set(x)
    zeros_acc = jnp.zeros((HALF, D), jnp.float32)

    seg_src, seg_dst, cnt, deg_rows = _sc_partition(
        srcf, dstf, fill_src, fill_dst, zeros_hist, iota_rows)
    seg_src = seg_src.reshape(NUM_CORES, 32, SEG_ROWS, 128)
    seg_dst = seg_dst.reshape(NUM_CORES, 32, SEG_ROWS, 128)
    deg_parts = jnp.broadcast_to(deg_rows.reshape(NUM_CORES, NPAD, 1),
                                 (NUM_CORES, NPAD, 16))
    hm1 = _tc_matmul(xp, W1)
    disb, hp1 = _tc_scale(deg_parts, hm1)

    acc1 = _sc_aggregate(hp1, seg_src, seg_dst, cnt, zeros_acc)
    hp2 = _tc_layer(acc1, hp1, disb, b1.reshape(1, D), W2)
    acc2 = _sc_aggregate(hp2, seg_src, seg_dst, cnt, zeros_acc)
    hp3 = _tc_layer(acc2, hp2, disb, b2.reshape(1, D), W3)
    acc3 = _sc_aggregate(hp3, seg_src, seg_dst, cnt, zeros_acc)
    out = _tc_final(acc3, hp3, disb, b3.reshape(1, D))
    return out[:N_NODES]


# speedup vs baseline: 25.0692x; 1.0676x over previous
import functools

import jax
import jax.numpy as jnp
from jax import lax
from jax.experimental import pallas as pl
from jax.experimental.pallas import tpu as pltpu
from jax.experimental.pallas import tpu_sc as plsc

N_NODES = 10000
D = 128
N_EDGES = 320000

NUM_CORES = 2
NUM_SUBCORES = 16

NPAD = 10240
EROWS = 2560
EPAD = EROWS * 128
HALF = NPAD // 2
ACC_ROWS = HALF + 8
TRASH = HALF

DEG_ROWS_PER_TILE = EROWS // (NUM_CORES * NUM_SUBCORES)
SEG = EPAD // (NUM_CORES * NUM_SUBCORES)
SEG_ROWS = SEG // 128
SEG_CHUNKS = SEG // 16
SEG_CAP_ROWS = 84
SEG_CAP = SEG_CAP_ROWS * 128
AGG_CHUNK = 128
AGG_EROWS = EPAD // AGG_CHUNK
AGG_ROWS_PER_TILE = AGG_EROWS // NUM_SUBCORES
AGG_NBUF = 2
DEG_WB = NPAD // NUM_SUBCORES
AGG_WB = HALF // NUM_SUBCORES

BLK = 1280
GRID = NPAD // BLK

_mesh = plsc.VectorSubcoreMesh(core_axis_name="c", subcore_axis_name="s")


@functools.partial(
    pl.kernel,
    out_type=[jax.ShapeDtypeStruct((NUM_CORES, 32, SEG_CAP), jnp.int32),
              jax.ShapeDtypeStruct((NUM_CORES, 32, SEG_CAP), jnp.int32),
              jax.ShapeDtypeStruct((NUM_CORES, 32, 16), jnp.int32),
              jax.ShapeDtypeStruct((NUM_CORES, NPAD // 128, 128), jnp.float32)],
    mesh=_mesh,
    scratch_types=[
        pltpu.VMEM((SEG,), jnp.int32),
        pltpu.VMEM((SEG,), jnp.int32),
        pltpu.VMEM((SEG_CAP,), jnp.int32),
        pltpu.VMEM((SEG_CAP,), jnp.int32),
        pltpu.VMEM((SEG_CAP,), jnp.int32),
        pltpu.VMEM((SEG_CAP,), jnp.int32),
        pltpu.VMEM((16,), jnp.int32),
        pltpu.VMEM((NPAD // 128, 128), jnp.float32),
        pltpu.VMEM((1, NPAD // 128), jnp.int32),
        pltpu.VMEM_SHARED((NPAD // 128, 128), jnp.float32),
    ],
    compiler_params=pltpu.CompilerParams(needs_layout_passes=False),
)
def _sc_partition(srcf_hbm, dstf_hbm, fsrc_hbm, fdst_hbm, z2_hbm, iota_hbm,
                  osrc_hbm, odst_hbm, ocnt_hbm, odeg_hbm,
                  srcv, dstv, cs0, cd0, cs1, cd1, cntv, degp, idv, deg_sh):
    cid = lax.axis_index("c")
    sid = lax.axis_index("s")
    t = cid * NUM_SUBCORES + sid
    pltpu.sync_copy(srcf_hbm.at[pl.ds(t * SEG, SEG)], srcv)
    pltpu.sync_copy(dstf_hbm.at[pl.ds(t * SEG, SEG)], dstv)
    pltpu.sync_copy(fsrc_hbm, cs0)
    pltpu.sync_copy(fsrc_hbm, cs1)
    pltpu.sync_copy(fdst_hbm, cd0)
    pltpu.sync_copy(fdst_hbm, cd1)
    pltpu.sync_copy(z2_hbm, degp)
    pltpu.sync_copy(iota_hbm, idv)
    @pl.when(sid < 10)
    def _():
        pltpu.sync_copy(z2_hbm.at[pl.ds(sid * 8, 8)],
                        deg_sh.at[pl.ds(sid * 8, 8)])
    plsc.subcore_barrier()

    ones_f = jnp.ones((16,), jnp.float32)

    def chunk(i, carry):
        o0, o1 = carry
        vs = srcv[pl.ds(i * 16, 16)]
        vd = dstv[pl.ds(i * 16, 16)]
        plsc.addupdate_scatter(degp, [vd >> 7, vd & 127], ones_f)
        m0 = vd < HALF
        plsc.store_compressed(cs0.at[pl.ds(o0, 16)], vs, mask=m0)
        plsc.store_compressed(cd0.at[pl.ds(o0, 16)], vd, mask=m0)
        m1 = jnp.logical_not(m0)
        plsc.store_compressed(cs1.at[pl.ds(o1, 16)], vs, mask=m1)
        plsc.store_compressed(cd1.at[pl.ds(o1, 16)], vd - HALF, mask=m1)
        n0 = jnp.sum(m0.astype(jnp.int32))
        return (o0 + n0, o1 + (16 - n0))

    o0, o1 = lax.fori_loop(0, SEG_CHUNKS, chunk, (0, 0))
    pltpu.sync_copy(cs0, osrc_hbm.at[0, t])
    pltpu.sync_copy(cd0, odst_hbm.at[0, t])
    pltpu.sync_copy(cs1, osrc_hbm.at[1, t])
    pltpu.sync_copy(cd1, odst_hbm.at[1, t])
    cntv[...] = jnp.full((16,), o0, jnp.int32)
    pltpu.sync_copy(cntv, ocnt_hbm.at[0, t])
    cntv[...] = jnp.full((16,), o1, jnp.int32)
    pltpu.sync_copy(cntv, ocnt_hbm.at[1, t])
    pltpu.sync_copy(degp, deg_sh.at[idv.at[0]], add=True)
    plsc.subcore_barrier()

    @pl.when(sid < 10)
    def _():
        pltpu.sync_copy(deg_sh.at[pl.ds(sid * 8, 8)],
                        odeg_hbm.at[cid, pl.ds(sid * 8, 8)])


@functools.partial(
    pl.kernel,
    out_type=jax.ShapeDtypeStruct((NUM_CORES, HALF, D), jnp.float32),
    mesh=_mesh,
    scratch_types=[
        pltpu.VMEM((SEG_CAP_ROWS, 128), jnp.int32),
        pltpu.VMEM((SEG_CAP_ROWS, 128), jnp.int32),
        pltpu.VMEM((16,), jnp.int32),
        pltpu.VMEM((AGG_CHUNK, D), jnp.float32),
        pltpu.VMEM((AGG_CHUNK, D), jnp.float32),
        pltpu.VMEM((AGG_CHUNK, D), jnp.float32),
        pltpu.VMEM_SHARED((ACC_ROWS, D), jnp.float32),
        pltpu.SemaphoreType.DMA,
        pltpu.SemaphoreType.DMA,
        pltpu.SemaphoreType.DMA,
        pltpu.SemaphoreType.DMA,
        pltpu.SemaphoreType.DMA,
        pltpu.SemaphoreType.DMA,
    ],
    compiler_params=pltpu.CompilerParams(needs_layout_passes=False),
)
def _sc_aggregate(hp_hbm, seg_src_hbm, seg_dst_hbm, cnt_hbm, z_hbm, out_hbm,
                  srci, dsti, cntv, buf0, buf1, buf2, acc_sh,
                  g0, g1, g2, s0, s1, s2):
    bufs = (buf0, buf1, buf2)
    gsems = (g0, g1, g2)
    ssems = (s0, s1, s2)
    cid = lax.axis_index("c")
    sid = lax.axis_index("s")
    wb = sid * AGG_WB
    pltpu.sync_copy(z_hbm.at[pl.ds(wb, AGG_WB)], acc_sh.at[pl.ds(wb, AGG_WB)])
    plsc.subcore_barrier()

    def run_segment(t):
        pltpu.sync_copy(seg_src_hbm.at[cid, t], srci)
        pltpu.sync_copy(seg_dst_hbm.at[cid, t], dsti)
        pltpu.sync_copy(cnt_hbm.at[cid, t], cntv)
        n = jnp.max(cntv[...])
        nr = (n + 127) // 128
        rows = jnp.minimum(jnp.maximum(((nr + 2) // 3) * 3, 3), SEG_CAP_ROWS)
        for b in range(3):
            pltpu.async_copy(hp_hbm.at[srci.at[b]], bufs[b], gsems[b])

        def step(i, _):
            c0 = 3 * i
            for b in range(3):
                j = c0 + b
                pltpu.make_async_copy(hp_hbm.at[srci.at[j]], bufs[b],
                                      gsems[b]).wait()
                pltpu.async_copy(bufs[b], acc_sh.at[dsti.at[j]], ssems[b],
                                 add=True)

                @pl.when(j + 3 < rows)
                def _():
                    pltpu.make_async_copy(bufs[b], acc_sh.at[dsti.at[j]],
                                          ssems[b]).wait()
                    pltpu.async_copy(hp_hbm.at[srci.at[j + 3]], bufs[b],
                                     gsems[b])
            return 0

        lax.fori_loop(0, rows // 3, step, 0)
        for b in range(3):
            pltpu.make_async_copy(bufs[b], acc_sh.at[dsti.at[b]],
                                  ssems[b]).wait()

    run_segment(2 * sid)
    run_segment(2 * sid + 1)
    plsc.subcore_barrier()
    pltpu.sync_copy(acc_sh.at[pl.ds(wb, AGG_WB)],
                    out_hbm.at[cid, pl.ds(wb, AGG_WB)])


def _tc_matmul(xp, W):
    def body(x_ref, w_ref, o_ref):
        o_ref[...] = jnp.dot(x_ref[...], w_ref[...],
                             preferred_element_type=jnp.float32)

    return pl.pallas_call(
        body,
        grid=(GRID,),
        in_specs=[pl.BlockSpec((BLK, D), lambda i: (i, 0)),
                  pl.BlockSpec((D, D), lambda i: (0, 0))],
        out_specs=pl.BlockSpec((BLK, D), lambda i: (i, 0)),
        out_shape=jax.ShapeDtypeStruct((NPAD, D), jnp.float32),
    )(xp, W)


def _tc_scale(deg_parts, hm):
    def body(dp_ref, hm_ref, disb_ref, hp_ref):
        deg = dp_ref[0] + dp_ref[1]
        dis = 1.0 / jnp.sqrt(deg[:, 0:1] + 1.0)
        disb = jnp.broadcast_to(dis, (BLK, D))
        disb_ref[...] = disb
        hp_ref[...] = disb * hm_ref[...]

    return pl.pallas_call(
        body,
        grid=(GRID,),
        in_specs=[pl.BlockSpec((NUM_CORES, BLK, 16), lambda i: (0, i, 0)),
                  pl.BlockSpec((BLK, D), lambda i: (i, 0))],
        out_specs=[pl.BlockSpec((BLK, D), lambda i: (i, 0)),
                   pl.BlockSpec((BLK, D), lambda i: (i, 0))],
        out_shape=[jax.ShapeDtypeStruct((NPAD, D), jnp.float32),
                   jax.ShapeDtypeStruct((NPAD, D), jnp.float32)],
    )(deg_parts, hm)


_ACC_SPEC = pl.BlockSpec((1, BLK, D), lambda i: (i // 4, i % 4, 0))


def _tc_layer(acc_parts, hp_prev, disb, bias, Wn):
    def body(a_ref, hp_ref, d_ref, b_ref, w_ref, o_ref):
        t = d_ref[...] * (a_ref[0] + hp_ref[...]) + b_ref[...]
        t = jnp.where(t >= 0, t, 0.01 * t)
        o_ref[...] = d_ref[...] * jnp.dot(t, w_ref[...],
                                          preferred_element_type=jnp.float32)

    return pl.pallas_call(
        body,
        grid=(GRID,),
        in_specs=[_ACC_SPEC,
                  pl.BlockSpec((BLK, D), lambda i: (i, 0)),
                  pl.BlockSpec((BLK, D), lambda i: (i, 0)),
                  pl.BlockSpec((1, D), lambda i: (0, 0)),
                  pl.BlockSpec((D, D), lambda i: (0, 0))],
        out_specs=pl.BlockSpec((BLK, D), lambda i: (i, 0)),
        out_shape=jax.ShapeDtypeStruct((NPAD, D), jnp.float32),
    )(acc_parts, hp_prev, disb, bias, Wn)


def _tc_final(acc_parts, hp_prev, disb, bias):
    def body(a_ref, hp_ref, d_ref, b_ref, o_ref):
        o_ref[...] = d_ref[...] * (a_ref[0] + hp_ref[...]) + b_ref[...]

    return pl.pallas_call(
        body,
        grid=(GRID,),
        in_specs=[_ACC_SPEC,
                  pl.BlockSpec((BLK, D), lambda i: (i, 0)),
                  pl.BlockSpec((BLK, D), lambda i: (i, 0)),
                  pl.BlockSpec((1, D), lambda i: (0, 0))],
        out_specs=pl.BlockSpec((BLK, D), lambda i: (i, 0)),
        out_shape=jax.ShapeDtypeStruct((NPAD, D), jnp.float32),
    )(acc_parts, hp_prev, disb, bias)


def kernel(x, edge_index, W1, b1, W2, b2, W3, b3):
    ei = edge_index.astype(jnp.int32)
    pad_src = jnp.arange(EPAD - N_EDGES, dtype=jnp.int32) % NPAD
    pad_dst = jnp.full((EPAD - N_EDGES,), N_NODES, jnp.int32)
    srcf = jnp.concatenate([ei[0], pad_src])
    dstf = jnp.concatenate([ei[1], pad_dst])
    fill_src = jnp.arange(SEG_CAP, dtype=jnp.int32) % NPAD
    fill_dst = jnp.full((SEG_CAP,), TRASH, jnp.int32)
    zeros_hist = jnp.zeros((NPAD // 128, 128), jnp.float32)
    iota_rows = jnp.arange(NPAD // 128, dtype=jnp.int32)[None, :]
    xp = jnp.zeros((NPAD, D), jnp.float32).at[:N_NODES].set(x)
    zeros_acc = jnp.zeros((HALF, D), jnp.float32)

    seg_src, seg_dst, cnt, deg_rows = _sc_partition(
        srcf, dstf, fill_src, fill_dst, zeros_hist, iota_rows)
    seg_src = seg_src.reshape(NUM_CORES, 32, SEG_CAP_ROWS, 128)
    seg_dst = seg_dst.reshape(NUM_CORES, 32, SEG_CAP_ROWS, 128)
    deg_parts = jnp.broadcast_to(deg_rows.reshape(NUM_CORES, NPAD, 1),
                                 (NUM_CORES, NPAD, 16))
    hm1 = _tc_matmul(xp, W1)
    disb, hp1 = _tc_scale(deg_parts, hm1)

    acc1 = _sc_aggregate(hp1, seg_src, seg_dst, cnt, zeros_acc)
    hp2 = _tc_layer(acc1, hp1, disb, b1.reshape(1, D), W2)
    acc2 = _sc_aggregate(hp2, seg_src, seg_dst, cnt, zeros_acc)
    hp3 = _tc_layer(acc2, hp2, disb, b2.reshape(1, D), W3)
    acc3 = _sc_aggregate(hp3, seg_src, seg_dst, cnt, zeros_acc)
    out = _tc_final(acc3, hp3, disb, b3.reshape(1, D))
    return out[:N_NODES]


# speedup vs baseline: 25.1056x; 1.0015x over previous
import functools

import jax
import jax.numpy as jnp
from jax import lax
from jax.experimental import pallas as pl
from jax.experimental.pallas import tpu as pltpu
from jax.experimental.pallas import tpu_sc as plsc

N_NODES = 10000
D = 128
N_EDGES = 320000

NUM_CORES = 2
NUM_SUBCORES = 16

NPAD = 10240
EPAD = 327680
HALF = NPAD // 2
ACC_ROWS = HALF + 8
TRASH = HALF

SEG = EPAD // (NUM_CORES * NUM_SUBCORES)
SEG_CHUNKS = SEG // 16
SEG_CAP_ROWS = 84
SEG_CAP = SEG_CAP_ROWS * 128
AGG_CHUNK = 128
HIST_ROWS = NPAD // 128
AGG_WB = HALF // NUM_SUBCORES

BLK = 1280
GRID = NPAD // BLK

_mesh = plsc.VectorSubcoreMesh(core_axis_name="c", subcore_axis_name="s")


@functools.partial(
    pl.kernel,
    out_type=[jax.ShapeDtypeStruct((NUM_CORES, 32, SEG_CAP), jnp.int32),
              jax.ShapeDtypeStruct((NUM_CORES, 32, SEG_CAP), jnp.int32),
              jax.ShapeDtypeStruct((NUM_CORES, 32, 16), jnp.int32),
              jax.ShapeDtypeStruct((NUM_CORES, HIST_ROWS, 128), jnp.float32)],
    mesh=_mesh,
    scratch_types=[
        pltpu.VMEM((SEG,), jnp.int32),
        pltpu.VMEM((SEG,), jnp.int32),
        pltpu.VMEM((SEG_CAP,), jnp.int32),
        pltpu.VMEM((SEG_CAP,), jnp.int32),
        pltpu.VMEM((SEG_CAP,), jnp.int32),
        pltpu.VMEM((SEG_CAP,), jnp.int32),
        pltpu.VMEM((16,), jnp.int32),
        pltpu.VMEM((HIST_ROWS, 128), jnp.float32),
        pltpu.VMEM((1, HIST_ROWS), jnp.int32),
        pltpu.VMEM_SHARED((HIST_ROWS, 128), jnp.float32),
    ],
    compiler_params=pltpu.CompilerParams(needs_layout_passes=False),
)
def _sc_partition(srcf_hbm, dstf_hbm, fsrc_hbm, fdst_hbm, z2_hbm, iota_hbm,
                  osrc_hbm, odst_hbm, ocnt_hbm, odeg_hbm,
                  srcv, dstv, cs0, cd0, cs1, cd1, cntv, degp, idv, deg_sh):
    cid = lax.axis_index("c")
    sid = lax.axis_index("s")
    t = cid * NUM_SUBCORES + sid
    pltpu.sync_copy(srcf_hbm.at[pl.ds(t * SEG, SEG)], srcv)
    pltpu.sync_copy(dstf_hbm.at[pl.ds(t * SEG, SEG)], dstv)
    pltpu.sync_copy(fsrc_hbm, cs0)
    pltpu.sync_copy(fsrc_hbm, cs1)
    pltpu.sync_copy(fdst_hbm, cd0)
    pltpu.sync_copy(fdst_hbm, cd1)
    pltpu.sync_copy(z2_hbm, degp)
    pltpu.sync_copy(iota_hbm, idv)
    @pl.when(sid < 10)
    def _():
        pltpu.sync_copy(z2_hbm.at[pl.ds(sid * 8, 8)],
                        deg_sh.at[pl.ds(sid * 8, 8)])
    plsc.subcore_barrier()

    ones_f = jnp.ones((16,), jnp.float32)

    def chunk(i, carry):
        o0, o1 = carry
        vs = srcv[pl.ds(i * 16, 16)]
        vd = dstv[pl.ds(i * 16, 16)]
        plsc.addupdate_scatter(degp, [vd >> 7, vd & 127], ones_f)
        m0 = vd < HALF
        plsc.store_compressed(cs0.at[pl.ds(o0, 16)], vs, mask=m0)
        plsc.store_compressed(cd0.at[pl.ds(o0, 16)], vd, mask=m0)
        m1 = jnp.logical_not(m0)
        plsc.store_compressed(cs1.at[pl.ds(o1, 16)], vs, mask=m1)
        plsc.store_compressed(cd1.at[pl.ds(o1, 16)], vd - HALF, mask=m1)
        n0 = jnp.sum(m0.astype(jnp.int32))
        return (o0 + n0, o1 + (16 - n0))

    o0, o1 = lax.fori_loop(0, SEG_CHUNKS, chunk, (0, 0))
    plsc.subcore_barrier()
    pltpu.sync_copy(cs0, osrc_hbm.at[0, t])
    pltpu.sync_copy(cd0, odst_hbm.at[0, t])
    pltpu.sync_copy(cs1, osrc_hbm.at[1, t])
    pltpu.sync_copy(cd1, odst_hbm.at[1, t])
    cntv[...] = jnp.full((16,), o0, jnp.int32)
    plsc.subcore_barrier()
    pltpu.sync_copy(cntv, ocnt_hbm.at[0, t])
    cntv[...] = jnp.full((16,), o1, jnp.int32)
    plsc.subcore_barrier()
    pltpu.sync_copy(cntv, ocnt_hbm.at[1, t])
    pltpu.sync_copy(degp, deg_sh.at[idv.at[0]], add=True)
    plsc.subcore_barrier()

    @pl.when(sid < 10)
    def _():
        pltpu.sync_copy(deg_sh.at[pl.ds(sid * 8, 8)],
                        odeg_hbm.at[cid, pl.ds(sid * 8, 8)])


@functools.partial(
    pl.kernel,
    out_type=jax.ShapeDtypeStruct((NUM_CORES, HALF, D), jnp.float32),
    mesh=_mesh,
    scratch_types=[
        pltpu.VMEM((SEG_CAP_ROWS, 128), jnp.int32),
        pltpu.VMEM((SEG_CAP_ROWS, 128), jnp.int32),
        pltpu.VMEM((16,), jnp.int32),
        pltpu.VMEM((AGG_CHUNK, D), jnp.float32),
        pltpu.VMEM((AGG_CHUNK, D), jnp.float32),
        pltpu.VMEM((AGG_CHUNK, D), jnp.float32),
        pltpu.VMEM_SHARED((ACC_ROWS, D), jnp.float32),
        pltpu.SemaphoreType.DMA,
        pltpu.SemaphoreType.DMA,
        pltpu.SemaphoreType.DMA,
        pltpu.SemaphoreType.DMA,
        pltpu.SemaphoreType.DMA,
        pltpu.SemaphoreType.DMA,
    ],
    compiler_params=pltpu.CompilerParams(needs_layout_passes=False),
)
def _sc_aggregate(hp_hbm, seg_src_hbm, seg_dst_hbm, cnt_hbm, z_hbm, out_hbm,
                  srci, dsti, cntv, buf0, buf1, buf2, acc_sh,
                  g0, g1, g2, s0, s1, s2):
    bufs = (buf0, buf1, buf2)
    gsems = (g0, g1, g2)
    ssems = (s0, s1, s2)
    cid = lax.axis_index("c")
    sid = lax.axis_index("s")
    wb = sid * AGG_WB
    pltpu.sync_copy(z_hbm.at[pl.ds(wb, AGG_WB)], acc_sh.at[pl.ds(wb, AGG_WB)])
    plsc.subcore_barrier()

    def run_segment(t):
        pltpu.sync_copy(seg_src_hbm.at[cid, t], srci)
        pltpu.sync_copy(seg_dst_hbm.at[cid, t], dsti)
        pltpu.sync_copy(cnt_hbm.at[cid, t], cntv)
        n = jnp.max(cntv[...])
        nr = (n + 127) // 128
        rows = jnp.minimum(jnp.maximum(((nr + 2) // 3) * 3, 3), SEG_CAP_ROWS)
        for b in range(3):
            pltpu.async_copy(hp_hbm.at[srci.at[b]], bufs[b], gsems[b])

        def step(i, _):
            c0 = 3 * i
            for b in range(3):
                j = c0 + b
                pltpu.make_async_copy(hp_hbm.at[srci.at[j]], bufs[b],
                                      gsems[b]).wait()
                pltpu.async_copy(bufs[b], acc_sh.at[dsti.at[j]], ssems[b],
                                 add=True)

                @pl.when(j + 3 < rows)
                def _():
                    pltpu.make_async_copy(bufs[b], acc_sh.at[dsti.at[j]],
                                          ssems[b]).wait()
                    pltpu.async_copy(hp_hbm.at[srci.at[j + 3]], bufs[b],
                                     gsems[b])
            return 0

        lax.fori_loop(0, rows // 3, step, 0)
        for b in range(3):
            pltpu.make_async_copy(bufs[b], acc_sh.at[dsti.at[b]],
                                  ssems[b]).wait()

    run_segment(2 * sid)
    run_segment(2 * sid + 1)
    plsc.subcore_barrier()
    pltpu.sync_copy(acc_sh.at[pl.ds(wb, AGG_WB)],
                    out_hbm.at[cid, pl.ds(wb, AGG_WB)])


def _tc_matmul(xp, W):
    def body(x_ref, w_ref, o_ref):
        o_ref[...] = jnp.dot(x_ref[...], w_ref[...],
                             preferred_element_type=jnp.float32)

    return pl.pallas_call(
        body,
        grid=(GRID,),
        in_specs=[pl.BlockSpec((BLK, D), lambda i: (i, 0)),
                  pl.BlockSpec((D, D), lambda i: (0, 0))],
        out_specs=pl.BlockSpec((BLK, D), lambda i: (i, 0)),
        out_shape=jax.ShapeDtypeStruct((NPAD, D), jnp.float32),
    )(xp, W)


def _tc_scale(deg_parts, hm):
    def body(dp_ref, hm_ref, disb_ref, hp_ref):
        deg = dp_ref[0] + dp_ref[1]
        dis = 1.0 / jnp.sqrt(deg[:, 0:1] + 1.0)
        disb = jnp.broadcast_to(dis, (BLK, D))
        disb_ref[...] = disb
        hp_ref[...] = disb * hm_ref[...]

    return pl.pallas_call(
        body,
        grid=(GRID,),
        in_specs=[pl.BlockSpec((NUM_CORES, BLK, 16), lambda i: (0, i, 0)),
                  pl.BlockSpec((BLK, D), lambda i: (i, 0))],
        out_specs=[pl.BlockSpec((BLK, D), lambda i: (i, 0)),
                   pl.BlockSpec((BLK, D), lambda i: (i, 0))],
        out_shape=[jax.ShapeDtypeStruct((NPAD, D), jnp.float32),
                   jax.ShapeDtypeStruct((NPAD, D), jnp.float32)],
    )(deg_parts, hm)


_ACC_SPEC = pl.BlockSpec((1, BLK, D), lambda i: (i // 4, i % 4, 0))


def _tc_layer(acc_parts, hp_prev, disb, bias, Wn):
    def body(a_ref, hp_ref, d_ref, b_ref, w_ref, o_ref):
        t = d_ref[...] * (a_ref[0] + hp_ref[...]) + b_ref[...]
        t = jnp.where(t >= 0, t, 0.01 * t)
        o_ref[...] = d_ref[...] * jnp.dot(t, w_ref[...],
                                          preferred_element_type=jnp.float32)

    return pl.pallas_call(
        body,
        grid=(GRID,),
        in_specs=[_ACC_SPEC,
                  pl.BlockSpec((BLK, D), lambda i: (i, 0)),
                  pl.BlockSpec((BLK, D), lambda i: (i, 0)),
                  pl.BlockSpec((1, D), lambda i: (0, 0)),
                  pl.BlockSpec((D, D), lambda i: (0, 0))],
        out_specs=pl.BlockSpec((BLK, D), lambda i: (i, 0)),
        out_shape=jax.ShapeDtypeStruct((NPAD, D), jnp.float32),
    )(acc_parts, hp_prev, disb, bias, Wn)


def _tc_final(acc_parts, hp_prev, disb, bias):
    def body(a_ref, hp_ref, d_ref, b_ref, o_ref):
        o_ref[...] = d_ref[...] * (a_ref[0] + hp_ref[...]) + b_ref[...]

    return pl.pallas_call(
        body,
        grid=(GRID,),
        in_specs=[_ACC_SPEC,
                  pl.BlockSpec((BLK, D), lambda i: (i, 0)),
                  pl.BlockSpec((BLK, D), lambda i: (i, 0)),
                  pl.BlockSpec((1, D), lambda i: (0, 0))],
        out_specs=pl.BlockSpec((BLK, D), lambda i: (i, 0)),
        out_shape=jax.ShapeDtypeStruct((NPAD, D), jnp.float32),
    )(acc_parts, hp_prev, disb, bias)


def kernel(x, edge_index, W1, b1, W2, b2, W3, b3):
    ei = edge_index.astype(jnp.int32)
    pad_src = jnp.arange(EPAD - N_EDGES, dtype=jnp.int32) % NPAD
    pad_dst = jnp.full((EPAD - N_EDGES,), N_NODES, jnp.int32)
    srcf = jnp.concatenate([ei[0], pad_src])
    dstf = jnp.concatenate([ei[1], pad_dst])
    fill_src = jnp.arange(SEG_CAP, dtype=jnp.int32) % NPAD
    fill_dst = jnp.full((SEG_CAP,), TRASH, jnp.int32)
    zeros_hist = jnp.zeros((HIST_ROWS, 128), jnp.float32)
    iota_rows = jnp.arange(HIST_ROWS, dtype=jnp.int32)[None, :]
    xp = jnp.zeros((NPAD, D), jnp.float32).at[:N_NODES].set(x)
    zeros_acc = jnp.zeros((HALF, D), jnp.float32)

    seg_src, seg_dst, cnt, deg_rows = _sc_partition(
        srcf, dstf, fill_src, fill_dst, zeros_hist, iota_rows)
    seg_src = seg_src.reshape(NUM_CORES, 32, SEG_CAP_ROWS, 128)
    seg_dst = seg_dst.reshape(NUM_CORES, 32, SEG_CAP_ROWS, 128)
    deg_parts = jnp.broadcast_to(deg_rows.reshape(NUM_CORES, NPAD, 1),
                                 (NUM_CORES, NPAD, 16))
    hm1 = _tc_matmul(xp, W1)
    disb, hp1 = _tc_scale(deg_parts, hm1)

    acc1 = _sc_aggregate(hp1, seg_src, seg_dst, cnt, zeros_acc)
    hp2 = _tc_layer(acc1, hp1, disb, b1.reshape(1, D), W2)
    acc2 = _sc_aggregate(hp2, seg_src, seg_dst, cnt, zeros_acc)
    hp3 = _tc_layer(acc2, hp2, disb, b2.reshape(1, D), W3)
    acc3 = _sc_aggregate(hp3, seg_src, seg_dst, cnt, zeros_acc)
    out = _tc_final(acc3, hp3, disb, b3.reshape(1, D))
    return out[:N_NODES]
